# baseline (speedup 1.0000x reference)
import jax
import jax.numpy as jnp
from jax import lax
from jax.experimental import pallas as pl
from jax.experimental.pallas import tpu as pltpu, tpu_sc as plsc

HIDDEN = 128

_NC = 2
_NS = 16
_NW = _NC * _NS

_SEQS = 4096
_SEQLEN = 50
_SEQ_PER_W = _SEQS // _NW
_NBUF = 5
_LOOK = 3


def _gather_body(table_hbm, idx_hbm, out_hbm, idx_v, rows_v, g_sem, w_sem):
    wid = lax.axis_index("s") * _NC + lax.axis_index("c")
    col = wid * _SEQ_PER_W
    pltpu.sync_copy(idx_hbm.at[:, pl.ds(col, _SEQ_PER_W)], idx_v)

    def start_gather(t, b):
        pltpu.async_copy(table_hbm.at[idx_v.at[t]], rows_v.at[b], g_sem.at[b])

    def wait_gather(b):
        pltpu.make_async_copy(
            table_hbm.at[idx_v.at[0]], rows_v.at[b], g_sem.at[b]
        ).wait()

    def start_write(t, b):
        pltpu.async_copy(
            rows_v.at[b], out_hbm.at[t, pl.ds(col, _SEQ_PER_W)], w_sem.at[b]
        )

    def wait_write(b):
        pltpu.make_async_copy(
            rows_v.at[b], out_hbm.at[0, pl.ds(col, _SEQ_PER_W)], w_sem.at[b]
        ).wait()

    def visit(t, has_prev_write, do_look):
        if has_prev_write:
            wait_write((t + _LOOK) % _NBUF)
        if do_look:
            start_gather(t + _LOOK, (t + _LOOK) % _NBUF)
        wait_gather(t % _NBUF)
        start_write(t, t % _NBUF)

    for t in range(_LOOK):
        start_gather(t, t % _NBUF)

    for t in range(_NBUF):
        visit(t, has_prev_write=(t >= _NBUF - _LOOK), do_look=True)

    def body(g, carry):
        for bi in range(_NBUF):
            t = g * _NBUF + bi
            visit(t, has_prev_write=True, do_look=True)
        return carry

    lax.fori_loop(1, _SEQLEN // _NBUF - 1, body, 0)

    for bi in range(_NBUF):
        t = (_SEQLEN // _NBUF - 1) * _NBUF + bi
        visit(t, has_prev_write=True, do_look=(t + _LOOK < _SEQLEN))

    for t in range(_SEQLEN - (_NBUF - _LOOK), _SEQLEN):
        wait_write(t % _NBUF)


def kernel(input_ids, table):
    ids_t = input_ids.T
    mesh = plsc.VectorSubcoreMesh(core_axis_name="c", subcore_axis_name="s")
    out_t = pl.kernel(
        _gather_body,
        mesh=mesh,
        out_type=jax.ShapeDtypeStruct((_SEQLEN, _SEQS, HIDDEN), jnp.float32),
        scratch_types=[
            pltpu.VMEM((_SEQLEN, _SEQ_PER_W), jnp.int32),
            pltpu.VMEM((_NBUF, _SEQ_PER_W, HIDDEN), jnp.float32),
            pltpu.SemaphoreType.DMA((_NBUF,)),
            pltpu.SemaphoreType.DMA((_NBUF,)),
        ],
    )(table, ids_t)
    return jnp.transpose(out_t, (1, 0, 2))
